# baseline (speedup 1.0000x reference)
import functools

import jax
import jax.numpy as jnp
from jax.experimental import pallas as pl
from jax.experimental.pallas import tpu as pltpu


def _fused_body(p_ref, w1_ref, b1_ref, w2f_ref, b2_ref, w3f_ref, b3_ref,
                cb_ref, cb2_ref, out_ref, *, T1, T2, T3, C, K):
    f32 = jnp.float32

    def down(hin, wf_ref, b_ref, t_out, relu):
        r = hin.reshape(t_out, 2, C)
        even = r[:, 0, :]
        odd = r[:, 1, :]
        zrow = jnp.zeros((1, C), f32)
        odd_r = jnp.concatenate([zrow, odd[:-1, :]], axis=0)
        even_l = jnp.concatenate([even[1:, :], zrow], axis=0)
        pat = jnp.concatenate([odd_r, even, odd, even_l], axis=1)
        acc = jnp.dot(pat, wf_ref[...], preferred_element_type=f32)
        acc = acc + b_ref[...]
        return jnp.maximum(acc, 0.0) if relu else acc

    h = jnp.dot(p_ref[0], w1_ref[...], preferred_element_type=f32)
    h = jnp.maximum(h + b1_ref[...], 0.0)
    h = down(h, w2f_ref, b2_ref, T2, relu=True)
    z = down(h, w3f_ref, b3_ref, T3, relu=False)

    cb = cb_ref[...]
    zc = jax.lax.dot_general(
        z, cb, (((1,), (1,)), ((), ())),
        preferred_element_type=f32)
    z2 = jnp.sum(z * z, axis=1, keepdims=True)
    d = z2 - 2.0 * zc + cb2_ref[...]
    minv = jnp.min(d, axis=1, keepdims=True)
    lane = jax.lax.broadcasted_iota(jnp.int32, (T3, K), 1)
    idx = jnp.min(jnp.where(d <= minv, lane, K), axis=1, keepdims=True)
    onehot = (lane == idx).astype(f32)
    q = jnp.dot(onehot, cb, preferred_element_type=f32,
                precision=jax.lax.Precision.HIGHEST)
    out_ref[0] = q


def kernel(x, w1, b1, w2, b2, w3, b3, codebook):
    B, _, T = x.shape
    C = w1.shape[0]
    K = codebook.shape[0]
    T1, T2, T3 = T // 2, T // 4, T // 8

    xp = jnp.pad(x[:, 0, :], ((0, 0), (1, 1)))
    patches = jnp.stack([xp[:, k::2][:, :T1] for k in range(4)], axis=-1)

    w1r = jnp.transpose(w1[:, 0, :])
    w2f = jnp.transpose(w2, (2, 1, 0)).reshape(4 * C, C)
    w3f = jnp.transpose(w3, (2, 1, 0)).reshape(4 * C, C)
    cb2 = jnp.sum(codebook * codebook, axis=1)[None, :]

    body = functools.partial(_fused_body, T1=T1, T2=T2, T3=T3, C=C, K=K)
    out = pl.pallas_call(
        body,
        grid=(B,),
        in_specs=[
            pl.BlockSpec((1, T1, 4), lambda b: (b, 0, 0)),
            pl.BlockSpec((4, C), lambda b: (0, 0)),
            pl.BlockSpec((1, C), lambda b: (0, 0)),
            pl.BlockSpec((4 * C, C), lambda b: (0, 0)),
            pl.BlockSpec((1, C), lambda b: (0, 0)),
            pl.BlockSpec((4 * C, C), lambda b: (0, 0)),
            pl.BlockSpec((1, C), lambda b: (0, 0)),
            pl.BlockSpec((K, C), lambda b: (0, 0)),
            pl.BlockSpec((1, K), lambda b: (0, 0)),
        ],
        out_specs=pl.BlockSpec((1, T3, C), lambda b: (b, 0, 0)),
        out_shape=jax.ShapeDtypeStruct((B, T3, C), jnp.float32),
        compiler_params=pltpu.CompilerParams(
            dimension_semantics=("arbitrary",)),
    )(patches, w1r, b1[None, :], w2f, b2[None, :], w3f, b3[None, :],
      codebook, cb2)
    return jnp.transpose(out, (0, 2, 1))


# speedup vs baseline: 1.2340x; 1.2340x over previous
import functools

import jax
import jax.numpy as jnp
from jax.experimental import pallas as pl
from jax.experimental.pallas import tpu as pltpu


def _fused_body(p_ref, w1_ref, b1_ref, w2f_ref, b2_ref, w3f_ref, b3_ref,
                cb_ref, cb2_ref, out_ref, *, T3, C, K):
    f32 = jnp.float32
    zrow = jnp.zeros((1, C), f32)

    hg = jnp.dot(p_ref[0], w1_ref[...], preferred_element_type=f32)
    hg = jnp.maximum(hg + b1_ref[...], 0.0)
    p0 = hg[0 * T3:1 * T3]
    p1 = hg[1 * T3:2 * T3]
    p2 = hg[2 * T3:3 * T3]
    p3 = hg[3 * T3:4 * T3]
    p3_r = jnp.concatenate([zrow, p3[:-1, :]], axis=0)
    p0_l = jnp.concatenate([p0[1:, :], zrow], axis=0)

    pat2 = jnp.concatenate(
        [jnp.concatenate([p3_r, p0, p1, p2], axis=1),
         jnp.concatenate([p1, p2, p3, p0_l], axis=1)], axis=0)
    h2 = jnp.dot(pat2, w2f_ref[...], preferred_element_type=f32)
    h2 = jnp.maximum(h2 + b2_ref[...], 0.0)
    he = h2[:T3]
    ho = h2[T3:]
    ho_r = jnp.concatenate([zrow, ho[:-1, :]], axis=0)
    he_l = jnp.concatenate([he[1:, :], zrow], axis=0)

    pat3 = jnp.concatenate([ho_r, he, ho, he_l], axis=1)
    z = jnp.dot(pat3, w3f_ref[...], preferred_element_type=f32)
    z = z + b3_ref[...]

    cb = cb_ref[...]
    zc = jax.lax.dot_general(
        z, cb, (((1,), (1,)), ((), ())),
        preferred_element_type=f32)
    z2 = jnp.sum(z * z, axis=1, keepdims=True)
    d = z2 - 2.0 * zc + cb2_ref[...]
    minv = jnp.min(d, axis=1, keepdims=True)
    lane = jax.lax.broadcasted_iota(jnp.int32, (T3, K), 1)
    idx = jnp.min(jnp.where(d <= minv, lane, K), axis=1, keepdims=True)
    onehot = (lane == idx).astype(f32)
    qt = jax.lax.dot_general(
        cb, onehot, (((0,), (1,)), ((), ())),
        preferred_element_type=f32,
        precision=jax.lax.Precision.HIGHEST)
    out_ref[0] = qt


def kernel(x, w1, b1, w2, b2, w3, b3, codebook):
    B, _, T = x.shape
    C = w1.shape[0]
    K = codebook.shape[0]
    T1, T3 = T // 2, T // 8

    xp = jnp.pad(x[:, 0, :], ((0, 0), (1, 1)))
    patches = jnp.stack([xp[:, k::2][:, :T1] for k in range(4)], axis=-1)
    patches = jnp.concatenate([patches[:, p::4, :] for p in range(4)],
                              axis=1)

    w1r = jnp.transpose(w1[:, 0, :])
    w2f = jnp.transpose(w2, (2, 1, 0)).reshape(4 * C, C)
    w3f = jnp.transpose(w3, (2, 1, 0)).reshape(4 * C, C)
    cb2 = jnp.sum(codebook * codebook, axis=1)[None, :]

    body = functools.partial(_fused_body, T3=T3, C=C, K=K)
    return pl.pallas_call(
        body,
        grid=(B,),
        in_specs=[
            pl.BlockSpec((1, T1, 4), lambda b: (b, 0, 0)),
            pl.BlockSpec((4, C), lambda b: (0, 0)),
            pl.BlockSpec((1, C), lambda b: (0, 0)),
            pl.BlockSpec((4 * C, C), lambda b: (0, 0)),
            pl.BlockSpec((1, C), lambda b: (0, 0)),
            pl.BlockSpec((4 * C, C), lambda b: (0, 0)),
            pl.BlockSpec((1, C), lambda b: (0, 0)),
            pl.BlockSpec((K, C), lambda b: (0, 0)),
            pl.BlockSpec((1, K), lambda b: (0, 0)),
        ],
        out_specs=pl.BlockSpec((1, C, T3), lambda b: (b, 0, 0)),
        out_shape=jax.ShapeDtypeStruct((B, C, T3), jnp.float32),
        compiler_params=pltpu.CompilerParams(
            dimension_semantics=("parallel",)),
    )(patches, w1r, b1[None, :], w2f, b2[None, :], w3f, b3[None, :],
      codebook, cb2)
